# baseline (speedup 1.0000x reference)
import jax
import jax.numpy as jnp
from jax import lax
from jax.experimental import pallas as pl
from jax.experimental.pallas import tpu as pltpu
from jax.experimental.pallas import tpu_sc as plsc

VOCAB = 30522
D = 768
S = 512
B = 128
N = S * B
EPS = 1e-12

NC = 2
NS = 16
NW = NC * NS
TOK_PER_W = N // NW
S_PER_W = S // NW
CHUNK = 32
NBUF = 4
CPS = B // CHUNK
NCHUNK = TOK_PER_W // CHUNK
NV = D // 16
HALF = NV // 2

def _lane_shuffle(x, idx):
    dnums = lax.GatherDimensionNumbers(
        offset_dims=(), collapsed_slice_dims=(0,), start_index_map=(0,))
    return lax.gather(x, idx[:, None], dnums, slice_sizes=(1,),
                      mode=lax.GatherScatterMode.PROMISE_IN_BOUNDS)


def _allreduce16(x):
    lanes = jnp.arange(16, dtype=jnp.int32)
    for k in (8, 4, 2, 1):
        x = x + _lane_shuffle(x, lanes ^ k)
    return x


def _rsqrt16(x):
    i = lax.bitcast_convert_type(x, jnp.int32)
    y = lax.bitcast_convert_type(jnp.int32(0x5F3759DF) - (i >> 1),
                                 jnp.float32)
    for _ in range(4):
        y = y * (1.5 - 0.5 * x * y * y)
    return y


def _fused_body(table, idx_hbm, pe_hbm, tok_hbm, gam_hbm, bet_hbm, out_hbm,
                idx_v, pe_v, tok_v, gam_v, bet_v, mean_v, rstd_v,
                buf0, buf1, buf2, buf3,
                g0, g1, g2, g3, o0, o1, o2, o3):
    wid = lax.axis_index("s") * NC + lax.axis_index("c")
    base = wid * TOK_PER_W
    s0 = wid * S_PER_W

    pltpu.sync_copy(idx_hbm.at[pl.ds(base, TOK_PER_W)], idx_v)
    pltpu.sync_copy(pe_hbm.at[pl.ds(s0, S_PER_W)], pe_v)
    pltpu.sync_copy(tok_hbm, tok_v)
    pltpu.sync_copy(gam_hbm, gam_v)
    pltpu.sync_copy(bet_hbm, bet_v)

    @plsc.parallel_loop(0, S_PER_W, step=1)
    def _(r):
        for j in range(NV):
            sl = pl.ds(j * 16, 16)
            pe_v[r, sl] = pe_v[r, sl] + tok_v[sl]

    bufs = (buf0, buf1, buf2, buf3)
    gsems = (g0, g1, g2, g3)
    osems = (o0, o1, o2, o3)

    def gather_desc(c, q):
        return pltpu.make_async_copy(
            table.at[idx_v.at[pl.ds(c * CHUNK, CHUNK)]], bufs[q], gsems[q])

    def out_desc(c, q):
        return pltpu.make_async_copy(
            bufs[q], out_hbm.at[pl.ds(base + c * CHUNK, CHUNK)], osems[q])

    gather_desc(0, 0).start()

    def sgroup(s, _):
        pe_regs = [pe_v[s, pl.ds(j * 16, 16)] for j in range(NV)]

        for q in range(NBUF):
            c = s * CPS + q
            buf = bufs[q]

            gather_desc(c, q).wait()

            nq = (q + 1) % NBUF

            @pl.when(c + 1 < NCHUNK)
            def _():
                @pl.when(c >= NBUF - 1)
                def _():
                    out_desc(c - (NBUF - 1), nq).wait()

                gather_desc(c + 1, nq).start()

            @plsc.parallel_loop(0, CHUNK, step=1)
            def _(r):
                acc_s = jnp.zeros((16,), jnp.float32)
                acc_q = jnp.zeros((16,), jnp.float32)
                for j in range(NV):
                    sl = pl.ds(j * 16, 16)
                    v = buf[r, sl] + pe_regs[j]
                    buf[r, sl] = v
                    acc_s = acc_s + v
                    acc_q = acc_q + v * v
                mean = _allreduce16(acc_s) * (1.0 / D)
                var = _allreduce16(acc_q) * (1.0 / D) - mean * mean
                var = jnp.maximum(var, 0.0) + EPS
                mean_v[r] = mean
                rstd_v[r] = _rsqrt16(var)

            for h in range(2):
                g_regs = [gam_v[pl.ds((h * HALF + j) * 16, 16)]
                          for j in range(HALF)]
                b_regs = [bet_v[pl.ds((h * HALF + j) * 16, 16)]
                          for j in range(HALF)]

                @plsc.parallel_loop(0, CHUNK, step=1)
                def _(r):
                    m = mean_v[r]
                    sd = rstd_v[r]
                    for j in range(HALF):
                        sl = pl.ds((h * HALF + j) * 16, 16)
                        v = buf[r, sl]
                        buf[r, sl] = (v - m) * sd * g_regs[j] + b_regs[j]

            out_desc(c, q).start()
        return 0

    lax.fori_loop(0, S_PER_W, sgroup, 0)

    for q in range(NBUF):
        out_desc(NCHUNK - NBUF + q, q).wait()


def _fused(word_emb, ids_flat, pe, tok_row, gamma, beta):
    mesh = plsc.VectorSubcoreMesh(core_axis_name="c", subcore_axis_name="s")
    fn = pl.kernel(
        _fused_body,
        out_type=jax.ShapeDtypeStruct((N, D), jnp.float32),
        mesh=mesh,
        scratch_types=[
            pltpu.VMEM((TOK_PER_W,), jnp.int32),
            pltpu.VMEM((S_PER_W, D), jnp.float32),
            pltpu.VMEM((D,), jnp.float32),
            pltpu.VMEM((D,), jnp.float32),
            pltpu.VMEM((D,), jnp.float32),
            pltpu.VMEM((CHUNK, 16), jnp.float32),
            pltpu.VMEM((CHUNK, 16), jnp.float32),
            pltpu.VMEM((CHUNK, D), jnp.float32),
            pltpu.VMEM((CHUNK, D), jnp.float32),
            pltpu.VMEM((CHUNK, D), jnp.float32),
            pltpu.VMEM((CHUNK, D), jnp.float32),
        ] + [pltpu.SemaphoreType.DMA] * 8,
    )
    return fn(word_emb, ids_flat, pe, tok_row, gamma, beta)


def kernel(input_ids, word_emb, pos_emb, tok_emb, ln_gamma, ln_beta):
    ids_flat = input_ids.reshape(N).astype(jnp.int32)
    tok_row = lax.slice(tok_emb, (0, 0), (1, D)).reshape(D)
    out = _fused(word_emb, ids_flat, pos_emb, tok_row,
                 ln_gamma.reshape(D), ln_beta.reshape(D))
    return out.reshape(S, B, D)


# speedup vs baseline: 1.0112x; 1.0112x over previous
import jax
import jax.numpy as jnp
from jax import lax
from jax.experimental import pallas as pl
from jax.experimental.pallas import tpu as pltpu
from jax.experimental.pallas import tpu_sc as plsc

VOCAB = 30522
D = 768
S = 512
B = 128
N = S * B
EPS = 1e-12

NC = 2
NS = 16
NW = NC * NS
TOK_PER_W = N // NW
S_PER_W = S // NW
CHUNK = 32
NBUF = 4
CPS = B // CHUNK
NCHUNK = TOK_PER_W // CHUNK
NV = D // 16
HALF = NV // 2

def _lane_shuffle(x, idx):
    dnums = lax.GatherDimensionNumbers(
        offset_dims=(), collapsed_slice_dims=(0,), start_index_map=(0,))
    return lax.gather(x, idx[:, None], dnums, slice_sizes=(1,),
                      mode=lax.GatherScatterMode.PROMISE_IN_BOUNDS)


def _allreduce16(x):
    lanes = jnp.arange(16, dtype=jnp.int32)
    for k in (8, 4, 2, 1):
        x = x + _lane_shuffle(x, lanes ^ k)
    return x


def _rsqrt16(x):
    i = lax.bitcast_convert_type(x, jnp.int32)
    y = lax.bitcast_convert_type(jnp.int32(0x5F3759DF) - (i >> 1),
                                 jnp.float32)
    for _ in range(4):
        y = y * (1.5 - 0.5 * x * y * y)
    return y


def _fused_body(table, idx_hbm, pe_hbm, tok_hbm, gam_hbm, bet_hbm, out_hbm,
                idx_v, pe_v, tok_v, gam_v, bet_v, mean_v, rstd_v,
                buf0, buf1, buf2, buf3,
                g0, g1, g2, g3, o0, o1, o2, o3):
    wid = lax.axis_index("s") * NC + lax.axis_index("c")
    base = wid * TOK_PER_W
    s0 = wid * S_PER_W

    pltpu.sync_copy(idx_hbm.at[pl.ds(base, TOK_PER_W)], idx_v)
    pltpu.sync_copy(pe_hbm.at[pl.ds(s0, S_PER_W)], pe_v)
    pltpu.sync_copy(tok_hbm, tok_v)
    pltpu.sync_copy(gam_hbm, gam_v)
    pltpu.sync_copy(bet_hbm, bet_v)

    @plsc.parallel_loop(0, S_PER_W, step=1)
    def _(r):
        for j in range(NV):
            sl = pl.ds(j * 16, 16)
            pe_v[r, sl] = pe_v[r, sl] + tok_v[sl]

    bufs = (buf0, buf1, buf2, buf3)
    gsems = (g0, g1, g2, g3)
    osems = (o0, o1, o2, o3)

    def gather_desc(c, q):
        return pltpu.make_async_copy(
            table.at[idx_v.at[pl.ds(c * CHUNK, CHUNK)]], bufs[q], gsems[q])

    def out_desc(c, q):
        return pltpu.make_async_copy(
            bufs[q], out_hbm.at[pl.ds(base + c * CHUNK, CHUNK)], osems[q])

    gather_desc(0, 0).start()

    def sgroup(s, _):
        for q in range(NBUF):
            c = s * CPS + q
            buf = bufs[q]

            gather_desc(c, q).wait()

            nq = (q + 1) % NBUF

            @pl.when(c + 1 < NCHUNK)
            def _():
                @pl.when(c >= NBUF - 1)
                def _():
                    out_desc(c - (NBUF - 1), nq).wait()

                gather_desc(c + 1, nq).start()

            @plsc.parallel_loop(0, CHUNK, step=1)
            def _(r):
                acc_s = jnp.zeros((16,), jnp.float32)
                acc_q = jnp.zeros((16,), jnp.float32)
                for j in range(NV):
                    sl = pl.ds(j * 16, 16)
                    v = buf[r, sl] + pe_v[s, sl]
                    buf[r, sl] = v
                    acc_s = acc_s + v
                    acc_q = acc_q + v * v
                mean = _allreduce16(acc_s) * (1.0 / D)
                var = _allreduce16(acc_q) * (1.0 / D) - mean * mean
                var = jnp.maximum(var, 0.0) + EPS
                mean_v[r] = mean
                rstd_v[r] = _rsqrt16(var)

            for h in range(2):
                g_regs = [gam_v[pl.ds((h * HALF + j) * 16, 16)]
                          for j in range(HALF)]
                b_regs = [bet_v[pl.ds((h * HALF + j) * 16, 16)]
                          for j in range(HALF)]

                @plsc.parallel_loop(0, CHUNK, step=1)
                def _(r):
                    m = mean_v[r]
                    sd = rstd_v[r]
                    for j in range(HALF):
                        sl = pl.ds((h * HALF + j) * 16, 16)
                        v = buf[r, sl]
                        buf[r, sl] = (v - m) * sd * g_regs[j] + b_regs[j]

            out_desc(c, q).start()
        return 0

    lax.fori_loop(0, S_PER_W, sgroup, 0)

    for q in range(NBUF):
        out_desc(NCHUNK - NBUF + q, q).wait()


def _fused(word_emb, ids_flat, pe, tok_row, gamma, beta):
    mesh = plsc.VectorSubcoreMesh(core_axis_name="c", subcore_axis_name="s")
    fn = pl.kernel(
        _fused_body,
        out_type=jax.ShapeDtypeStruct((N, D), jnp.float32),
        mesh=mesh,
        scratch_types=[
            pltpu.VMEM((TOK_PER_W,), jnp.int32),
            pltpu.VMEM((S_PER_W, D), jnp.float32),
            pltpu.VMEM((D,), jnp.float32),
            pltpu.VMEM((D,), jnp.float32),
            pltpu.VMEM((D,), jnp.float32),
            pltpu.VMEM((CHUNK, 16), jnp.float32),
            pltpu.VMEM((CHUNK, 16), jnp.float32),
            pltpu.VMEM((CHUNK, D), jnp.float32),
            pltpu.VMEM((CHUNK, D), jnp.float32),
            pltpu.VMEM((CHUNK, D), jnp.float32),
            pltpu.VMEM((CHUNK, D), jnp.float32),
        ] + [pltpu.SemaphoreType.DMA] * 8,
    )
    return fn(word_emb, ids_flat, pe, tok_row, gamma, beta)


def kernel(input_ids, word_emb, pos_emb, tok_emb, ln_gamma, ln_beta):
    ids_flat = input_ids.reshape(N).astype(jnp.int32)
    tok_row = lax.slice(tok_emb, (0, 0), (1, D)).reshape(D)
    out = _fused(word_emb, ids_flat, pos_emb, tok_row,
                 ln_gamma.reshape(D), ln_beta.reshape(D))
    return out.reshape(S, B, D)


# speedup vs baseline: 1.4125x; 1.3969x over previous
import jax
import jax.numpy as jnp
from jax import lax
from jax.experimental import pallas as pl
from jax.experimental.pallas import tpu as pltpu
from jax.experimental.pallas import tpu_sc as plsc

VOCAB = 30522
D = 768
S = 512
B = 128
N = S * B
EPS = 1e-12

NC = 2
NS = 16
NW = NC * NS

K = 2
SC_CHUNK = S // K
NK = SC_CHUNK * B
TOK_PER_W = NK // NW
CHUNK = 64
NCHUNK = TOK_PER_W // CHUNK
NBUF = 2


def _sc_gather_body(table, idx_hbm, out_hbm, idx_v, rows0, rows1,
                    gsem0, gsem1, osem0, osem1):
    wid = lax.axis_index("s") * NC + lax.axis_index("c")
    base = wid * TOK_PER_W
    pltpu.sync_copy(idx_hbm.at[pl.ds(base, TOK_PER_W)], idx_v)

    bufs = (rows0, rows1)
    gsems = (gsem0, gsem1)
    osems = (osem0, osem1)

    def gather_desc(g, b):
        return pltpu.make_async_copy(
            table.at[idx_v.at[pl.ds(g * CHUNK, CHUNK)]], bufs[b], gsems[b])

    def out_desc(g, b):
        return pltpu.make_async_copy(
            bufs[b], out_hbm.at[pl.ds(base + g * CHUNK, CHUNK)], osems[b])

    def step(g2, _):
        for b in range(NBUF):
            g = g2 * NBUF + b

            @pl.when(g >= NBUF)
            def _():
                out_desc(g - NBUF, b).wait()

            d = gather_desc(g, b)
            d.start()
            d.wait()
            out_desc(g, b).start()
        return 0

    lax.fori_loop(0, NCHUNK // NBUF, step, 0)
    for b in range(NBUF):
        out_desc(NCHUNK - NBUF + b, b).wait()


def _sc_gather(word_emb, ids_chunk):
    mesh = plsc.VectorSubcoreMesh(core_axis_name="c", subcore_axis_name="s")
    fn = pl.kernel(
        _sc_gather_body,
        out_type=jax.ShapeDtypeStruct((NK, D), jnp.float32),
        mesh=mesh,
        scratch_types=[
            pltpu.VMEM((TOK_PER_W,), jnp.int32),
            pltpu.VMEM((CHUNK, D), jnp.float32),
            pltpu.VMEM((CHUNK, D), jnp.float32),
            pltpu.SemaphoreType.DMA,
            pltpu.SemaphoreType.DMA,
            pltpu.SemaphoreType.DMA,
            pltpu.SemaphoreType.DMA,
        ],
    )
    return fn(word_emb, ids_chunk)


SBLK = 16
STEPS_PER_K = SC_CHUNK // SBLK


def _ln_compute(g_ref, pe_ref, te_ref, gamma_ref, beta_ref, o_ref):
    emb = g_ref[...] + pe_ref[...][:, None, :] + te_ref[...][None, :, :]
    mean = jnp.mean(emb, axis=-1, keepdims=True)
    cen = emb - mean
    var = jnp.mean(cen * cen, axis=-1, keepdims=True)
    o_ref[...] = (cen * lax.rsqrt(var + EPS)) * gamma_ref[...] + beta_ref[...]


def _ln_body_first(g_ref, pe_ref, te_ref, gamma_ref, beta_ref, o_ref):
    _ln_compute(g_ref, pe_ref, te_ref, gamma_ref, beta_ref, o_ref)


def _ln_body_chain(g_ref, pe_ref, te_ref, gamma_ref, beta_ref, prev_ref,
                   o_ref):
    del prev_ref
    _ln_compute(g_ref, pe_ref, te_ref, gamma_ref, beta_ref, o_ref)


def _ln_chunk(k, gathered_k, pe_k, tok_row, ln_gamma, ln_beta, prev):
    base = k * STEPS_PER_K
    out_spec = pl.BlockSpec((SBLK, B, D), lambda i: (base + i, 0, 0))
    in_specs = [
        pl.BlockSpec((SBLK, B, D), lambda i: (i, 0, 0)),
        pl.BlockSpec((SBLK, D), lambda i: (i, 0)),
        pl.BlockSpec((1, D), lambda i: (0, 0)),
        pl.BlockSpec((1, D), lambda i: (0, 0)),
        pl.BlockSpec((1, D), lambda i: (0, 0)),
    ]
    args = [gathered_k, pe_k, tok_row, ln_gamma, ln_beta]
    if prev is None:
        body = _ln_body_first
        aliases = {}
    else:
        body = _ln_body_chain
        in_specs.append(pl.BlockSpec(memory_space=pl.ANY))
        args.append(prev)
        aliases = {5: 0}
    return pl.pallas_call(
        body,
        grid=(STEPS_PER_K,),
        in_specs=in_specs,
        out_specs=out_spec,
        out_shape=jax.ShapeDtypeStruct((S, B, D), jnp.float32),
        input_output_aliases=aliases,
    )(*args)


def kernel(input_ids, word_emb, pos_emb, tok_emb, ln_gamma, ln_beta):
    ids_flat = input_ids.reshape(N).astype(jnp.int32)
    tok_row = lax.slice(tok_emb, (0, 0), (1, D))
    gamma = ln_gamma.reshape(1, D)
    beta = ln_beta.reshape(1, D)

    gathered = [
        _sc_gather(word_emb, lax.slice(ids_flat, (k * NK,), ((k + 1) * NK,)))
        for k in range(K)
    ]
    out = None
    for k in range(K):
        pe_k = lax.slice(pos_emb, (k * SC_CHUNK, 0), ((k + 1) * SC_CHUNK, D))
        out = _ln_chunk(k, gathered[k].reshape(SC_CHUNK, B, D), pe_k,
                        tok_row, gamma, beta, out)
    return out
